# baseline (speedup 1.0000x reference)
import functools

import jax
import jax.numpy as jnp
from jax import lax
from jax.experimental import pallas as pl
from jax.experimental.pallas import tpu as pltpu
from jax.experimental.pallas import tpu_sc as plsc

_B = 16384
_N_NUM = 13
_N_CAT = 26
_VOCAB = 100000
_D = 16
_L = 16

_NC = 2
_NS = 16
_NW = _NC * _NS

_BPW = _B // _NW
_NB = 128
_NCHUNK = _BPW // _NB


def _sc_body(xnum_hbm, xcat_hbm, w_hbm, b_hbm, tab_hbm, out_hbm,
             cidx, coidx, crow, noidx, nrow, xnum_v, wv, bv,
             sem_g, sem_n, sem_c):
    wid = lax.axis_index("s") * _NC + lax.axis_index("c")
    b0 = wid * _BPW

    pltpu.sync_copy(w_hbm, wv)
    pltpu.sync_copy(b_hbm, bv)

    iota = lax.iota(jnp.int32, _L)

    def chunk(ci, _):
        bc = b0 + ci * _NB

        pltpu.sync_copy(xcat_hbm.at[pl.ds(bc * _N_CAT, _NB * _N_CAT)], cidx)

        def cgrp(g, _):
            base = g * _L
            j = base + iota
            f = lax.rem(j, _N_CAT)
            v = cidx[pl.ds(base, _L)]
            cidx[pl.ds(base, _L)] = v + f * _VOCAB
            bglob = bc + lax.div(j, _N_CAT)
            coidx[pl.ds(base, _L)] = bglob * (_N_NUM + _N_CAT) + _N_NUM + f
            return 0

        lax.fori_loop(0, _NB * _N_CAT // _L, cgrp, 0)

        gcopy = pltpu.async_copy(tab_hbm.at[cidx], crow, sem_g)

        pltpu.sync_copy(xnum_hbm.at[pl.ds(bc * _N_NUM, _NB * _N_NUM)], xnum_v)

        def ngrp(g, _):
            base = g * _L
            j = base + iota
            f = lax.rem(j, _N_NUM)
            bglob = bc + lax.div(j, _N_NUM)
            noidx[pl.ds(base, _L)] = bglob * (_N_NUM + _N_CAT) + f
            return 0

        lax.fori_loop(0, _NB * _N_NUM // _L, ngrp, 0)

        def nbatch(bl, _):
            rowbase = bl * _N_NUM
            for f in range(_N_NUM):
                sel = lax.broadcast_in_dim(rowbase + f, (_L,), ())
                xv = plsc.load_gather(xnum_v, [sel])
                nrow[rowbase + f, :] = xv * wv[f, :] + bv[f, :]
            return 0

        lax.fori_loop(0, _NB, nbatch, 0)

        ncopy = pltpu.async_copy(nrow, out_hbm.at[noidx], sem_n)
        gcopy.wait()
        ccopy = pltpu.async_copy(crow, out_hbm.at[coidx], sem_c)
        ncopy.wait()
        ccopy.wait()
        return 0

    lax.fori_loop(0, _NCHUNK, chunk, 0)


@functools.partial(jax.jit, static_argnames=())
def _sc_tokenize(xnum_flat, xcat_flat, W_num, b_num, tab_flat):
    mesh = plsc.VectorSubcoreMesh(core_axis_name="c", subcore_axis_name="s")
    k = pl.kernel(
        _sc_body,
        out_type=jax.ShapeDtypeStruct((_B * (_N_NUM + _N_CAT), _D), jnp.float32),
        mesh=mesh,
        compiler_params=pltpu.CompilerParams(
            needs_layout_passes=False, use_tc_tiling_on_sc=False),
        scratch_types=[
            pltpu.VMEM((_NB * _N_CAT,), jnp.int32),
            pltpu.VMEM((_NB * _N_CAT,), jnp.int32),
            pltpu.VMEM((_NB * _N_CAT, _D), jnp.float32),
            pltpu.VMEM((_NB * _N_NUM,), jnp.int32),
            pltpu.VMEM((_NB * _N_NUM, _D), jnp.float32),
            pltpu.VMEM((_NB * _N_NUM,), jnp.float32),
            pltpu.VMEM((_N_NUM, _D), jnp.float32),
            pltpu.VMEM((_N_NUM, _D), jnp.float32),
            pltpu.SemaphoreType.DMA,
            pltpu.SemaphoreType.DMA,
            pltpu.SemaphoreType.DMA,
        ],
    )
    return k(xnum_flat, xcat_flat, W_num, b_num, tab_flat)


def kernel(x_num, x_cat, W_num, b_num, tables):
    xnum_flat = x_num.reshape(-1)
    xcat_flat = x_cat.reshape(-1).astype(jnp.int32)
    tab_flat = tables.reshape(_N_CAT * _VOCAB, _D)
    out_flat = _sc_tokenize(xnum_flat, xcat_flat, W_num, b_num, tab_flat)
    return out_flat.reshape(_B, _N_NUM + _N_CAT, _D)


# speedup vs baseline: 6.8009x; 6.8009x over previous
import functools

import jax
import jax.numpy as jnp
from jax import lax
from jax.experimental import pallas as pl
from jax.experimental.pallas import tpu as pltpu
from jax.experimental.pallas import tpu_sc as plsc

_B = 16384
_N_NUM = 13
_N_CAT = 26
_VOCAB = 100000
_D = 16
_L = 16
_TOK = _N_NUM + _N_CAT

_NC = 2
_NS = 16
_NW = _NC * _NS

_NUM_JOBS = _N_NUM * _D
_ALL_JOBS = _TOK * _D
_MAX_JOBS_PER_W = -(-_ALL_JOBS // _NW)

_BC = 8192
_NCH = _B // _BC


def _sc_body(xnum_hbm, xcat_hbm, wflat_hbm, bflat_hbm, tab_hbm, out_hbm,
             tslice, idxb, obuf, xbuf, wv, bv):
    wid = lax.axis_index("s") * _NC + lax.axis_index("c")

    pltpu.sync_copy(wflat_hbm, wv)
    pltpu.sync_copy(bflat_hbm, bv)

    def num_job(p):
        f = lax.div(p, _D)
        d = lax.rem(p, _D)
        sel = lax.broadcast_in_dim(p, (_L,), ())
        ws = plsc.load_gather(wv, [sel])
        bs = plsc.load_gather(bv, [sel])
        for c in range(_NCH):
            pltpu.sync_copy(xnum_hbm.at[f, pl.ds(c * _BC, _BC)], xbuf)

            def grp(g, _):
                base = g * _L
                obuf[pl.ds(base, _L)] = xbuf[pl.ds(base, _L)] * ws + bs
                return 0

            lax.fori_loop(0, _BC // _L, grp, 0)
            pltpu.sync_copy(obuf, out_hbm.at[f, d, pl.ds(c * _BC, _BC)])

    def cat_job(p):
        j = p - _NUM_JOBS
        f = lax.div(j, _D)
        d = lax.rem(j, _D)
        pltpu.sync_copy(tab_hbm.at[f, d, :], tslice)
        for c in range(_NCH):
            pltpu.sync_copy(xcat_hbm.at[f, pl.ds(c * _BC, _BC)], idxb)

            def grp(g, _):
                base = g * _L
                iv = idxb[pl.ds(base, _L)]
                obuf[pl.ds(base, _L)] = plsc.load_gather(tslice, [iv])
                return 0

            lax.fori_loop(0, _BC // _L, grp, 0)
            pltpu.sync_copy(obuf, out_hbm.at[_N_NUM + f, d, pl.ds(c * _BC, _BC)])

    for k in range(_MAX_JOBS_PER_W):
        p = wid + _NW * k

        @pl.when(p < _NUM_JOBS)
        def _():
            num_job(p)

        @pl.when(jnp.logical_and(p >= _NUM_JOBS, p < _ALL_JOBS))
        def _():
            cat_job(p)


@jax.jit
def _sc_tokenize(xnum_t, xcat_t, wflat, bflat, tab_t):
    mesh = plsc.VectorSubcoreMesh(core_axis_name="c", subcore_axis_name="s")
    k = pl.kernel(
        _sc_body,
        out_type=jax.ShapeDtypeStruct((_TOK, _D, _B), jnp.float32),
        mesh=mesh,
        compiler_params=pltpu.CompilerParams(needs_layout_passes=False),
        scratch_types=[
            pltpu.VMEM((_VOCAB,), jnp.float32),
            pltpu.VMEM((_BC,), jnp.int32),
            pltpu.VMEM((_BC,), jnp.float32),
            pltpu.VMEM((_BC,), jnp.float32),
            pltpu.VMEM((_NUM_JOBS,), jnp.float32),
            pltpu.VMEM((_NUM_JOBS,), jnp.float32),
        ],
    )
    return k(xnum_t, xcat_t, wflat, bflat, tab_t)


def kernel(x_num, x_cat, W_num, b_num, tables):
    xnum_t = x_num.T
    xcat_t = x_cat.T.astype(jnp.int32)
    tab_t = jnp.transpose(tables, (0, 2, 1))
    wflat = W_num.reshape(-1)
    bflat = b_num.reshape(-1)
    out_t = _sc_tokenize(xnum_t, xcat_t, wflat, bflat, tab_t)
    return jnp.transpose(out_t, (2, 0, 1))


# speedup vs baseline: 9.5184x; 1.3996x over previous
import functools

import jax
import jax.numpy as jnp
from jax import lax
from jax.experimental import pallas as pl
from jax.experimental.pallas import tpu as pltpu
from jax.experimental.pallas import tpu_sc as plsc

_B = 16384
_N_NUM = 13
_N_CAT = 26
_VOCAB = 100000
_D = 16
_L = 16
_TOK = _N_NUM + _N_CAT

_NC = 2
_NS = 16
_NW = _NC * _NS

_NUM_JOBS = _N_NUM * _D
_ALL_JOBS = _TOK * _D
_MAX_JOBS_PER_W = -(-_ALL_JOBS // _NW)

_BC = 4096
_NCH = _B // _BC
_GRP = _BC // _L


def _sc_body(xnum_hbm, xcat_hbm, wflat_hbm, bflat_hbm, tab_hbm, out_hbm,
             tslice, idxb, obuf, xbuf, wv, bv,
             sem_t, sem_i, sem_o0, sem_o1):
    wid = lax.axis_index("s") * _NC + lax.axis_index("c")

    pltpu.sync_copy(wflat_hbm, wv)
    pltpu.sync_copy(bflat_hbm, bv)

    def num_job(p):
        f = lax.div(p, _D)
        d = lax.rem(p, _D)
        sel = lax.broadcast_in_dim(p, (_L,), ())
        ws = plsc.load_gather(wv, [sel])
        bs = plsc.load_gather(bv, [sel])
        for c in range(_NCH):
            sl = c % 2
            sem_o = sem_o0 if sl == 0 else sem_o1
            pltpu.sync_copy(xnum_hbm.at[f, pl.ds(c * _BC, _BC)], xbuf)
            if c >= 2:
                pltpu.make_async_copy(obuf.at[sl], out_hbm.at[f, d, pl.ds(0, _BC)],
                                      sem_o).wait()

            @plsc.parallel_loop(0, _GRP, unroll=4)
            def _(g):
                base = g * _L
                obuf[sl, pl.ds(base, _L)] = xbuf[pl.ds(base, _L)] * ws + bs

            pltpu.async_copy(obuf.at[sl], out_hbm.at[f, d, pl.ds(c * _BC, _BC)],
                             sem_o)
        pltpu.make_async_copy(obuf.at[0], out_hbm.at[f, d, pl.ds(0, _BC)],
                              sem_o0).wait()
        pltpu.make_async_copy(obuf.at[1], out_hbm.at[f, d, pl.ds(0, _BC)],
                              sem_o1).wait()

    def cat_job(p):
        j = p - _NUM_JOBS
        f = lax.div(j, _D)
        d = lax.rem(j, _D)
        tcp = pltpu.async_copy(tab_hbm.at[f, d, :], tslice, sem_t)
        icp = pltpu.async_copy(xcat_hbm.at[f, :], idxb, sem_i)
        tcp.wait()
        icp.wait()
        for c in range(_NCH):
            sl = c % 2
            sem_o = sem_o0 if sl == 0 else sem_o1
            if c >= 2:
                pltpu.make_async_copy(obuf.at[sl], out_hbm.at[f, d, pl.ds(0, _BC)],
                                      sem_o).wait()
            cbase = c * _BC

            @plsc.parallel_loop(0, _GRP, unroll=4)
            def _(g):
                base = g * _L
                iv = idxb[pl.ds(cbase + base, _L)]
                obuf[sl, pl.ds(base, _L)] = plsc.load_gather(tslice, [iv])

            pltpu.async_copy(obuf.at[sl],
                             out_hbm.at[_N_NUM + f, d, pl.ds(cbase, _BC)], sem_o)
        pltpu.make_async_copy(obuf.at[0], out_hbm.at[f, d, pl.ds(0, _BC)],
                              sem_o0).wait()
        pltpu.make_async_copy(obuf.at[1], out_hbm.at[f, d, pl.ds(0, _BC)],
                              sem_o1).wait()

    for k in range(_MAX_JOBS_PER_W):
        p = wid + _NW * k

        @pl.when(p < _NUM_JOBS)
        def _():
            num_job(p)

        @pl.when(jnp.logical_and(p >= _NUM_JOBS, p < _ALL_JOBS))
        def _():
            cat_job(p)


@jax.jit
def _sc_tokenize(xnum_t, xcat_t, wflat, bflat, tab_t):
    mesh = plsc.VectorSubcoreMesh(core_axis_name="c", subcore_axis_name="s")
    k = pl.kernel(
        _sc_body,
        out_type=jax.ShapeDtypeStruct((_TOK, _D, _B), jnp.float32),
        mesh=mesh,
        compiler_params=pltpu.CompilerParams(needs_layout_passes=False),
        scratch_types=[
            pltpu.VMEM((_VOCAB,), jnp.float32),
            pltpu.VMEM((_B,), jnp.int32),
            pltpu.VMEM((2, _BC), jnp.float32),
            pltpu.VMEM((_BC,), jnp.float32),
            pltpu.VMEM((_NUM_JOBS,), jnp.float32),
            pltpu.VMEM((_NUM_JOBS,), jnp.float32),
            pltpu.SemaphoreType.DMA,
            pltpu.SemaphoreType.DMA,
            pltpu.SemaphoreType.DMA,
            pltpu.SemaphoreType.DMA,
        ],
    )
    return k(xnum_t, xcat_t, wflat, bflat, tab_t)


def kernel(x_num, x_cat, W_num, b_num, tables):
    xnum_t = x_num.T
    xcat_t = x_cat.T.astype(jnp.int32)
    tab_t = jnp.transpose(tables, (0, 2, 1))
    wflat = W_num.reshape(-1)
    bflat = b_num.reshape(-1)
    out_t = _sc_tokenize(xnum_t, xcat_t, wflat, bflat, tab_t)
    return jnp.transpose(out_t, (2, 0, 1))


# speedup vs baseline: 11.9381x; 1.2542x over previous
import jax
import jax.numpy as jnp
from jax import lax
from jax.experimental import pallas as pl
from jax.experimental.pallas import tpu as pltpu
from jax.experimental.pallas import tpu_sc as plsc

_B = 16384
_N_NUM = 13
_N_CAT = 26
_VOCAB = 100000
_D = 16
_L = 16
_TOK = _N_NUM + _N_CAT

_NC = 2
_NS = 16
_NW = _NC * _NS

_CAT_JOBS_PER_W = (_N_CAT * _D) // _NW
_NUM_JOBS = _N_NUM * _D

_BC = 4096
_NCH = _B // _BC
_GRP = _BC // _L
_NBC = 2048
_NNCH = _B // _NBC
_NGRP = _NBC // _L


def _sc_body(xnum_hbm, xcat_hbm, wflat_hbm, bflat_hbm, tab_hbm, out_hbm,
             tslice, idxb, obuf, nbuf, wv, bv,
             sem_t, sem_i, sem_o0, sem_o1, sem_n0, sem_n1):
    wid = lax.axis_index("s") * _NC + lax.axis_index("c")

    pltpu.sync_copy(wflat_hbm, wv)
    pltpu.sync_copy(bflat_hbm, bv)

    n0 = lax.div(13 * wid, 2)
    n1 = lax.div(13 * (wid + 1), 2)

    def num_job(q):
        f = lax.div(q, _D)
        d = lax.rem(q, _D)
        sel = lax.broadcast_in_dim(q, (_L,), ())
        ws = plsc.load_gather(wv, [sel])
        bs = plsc.load_gather(bv, [sel])
        for c in range(_NNCH):
            sl = c % 2
            sem_n = sem_n0 if sl == 0 else sem_n1
            if c >= 2:
                pltpu.make_async_copy(nbuf.at[sl], out_hbm.at[f, d, pl.ds(0, _NBC)],
                                      sem_n).wait()
            pltpu.sync_copy(xnum_hbm.at[f, pl.ds(c * _NBC, _NBC)], nbuf.at[sl])

            @plsc.parallel_loop(0, _NGRP, unroll=4)
            def _(g):
                base = g * _L
                nbuf[sl, pl.ds(base, _L)] = nbuf[sl, pl.ds(base, _L)] * ws + bs

            pltpu.async_copy(nbuf.at[sl], out_hbm.at[f, d, pl.ds(c * _NBC, _NBC)],
                             sem_n)
        pltpu.make_async_copy(nbuf.at[0], out_hbm.at[0, 0, pl.ds(0, _NBC)],
                              sem_n0).wait()
        pltpu.make_async_copy(nbuf.at[1], out_hbm.at[0, 0, pl.ds(0, _NBC)],
                              sem_n1).wait()

    def cat_iter(i, prev_f):
        j = _CAT_JOBS_PER_W * wid + i
        f = lax.div(j, _D)
        d = lax.rem(j, _D)

        pltpu.async_copy(tab_hbm.at[f, d, :], tslice, sem_t)
        new_f = f != prev_f

        @pl.when(new_f)
        def _():
            pltpu.async_copy(xcat_hbm.at[f, :], idxb, sem_i)

        q = n0 + i

        @pl.when(q < n1)
        def _():
            num_job(q)

        @pl.when(new_f)
        def _():
            pltpu.make_async_copy(xcat_hbm.at[f, :], idxb, sem_i).wait()

        pltpu.make_async_copy(tab_hbm.at[f, d, :], tslice, sem_t).wait()

        for c in range(_NCH):
            sl = c % 2
            sem_o = sem_o0 if sl == 0 else sem_o1
            if c >= 2:
                pltpu.make_async_copy(obuf.at[sl], out_hbm.at[0, 0, pl.ds(0, _BC)],
                                      sem_o).wait()
            cbase = c * _BC

            @plsc.parallel_loop(0, _GRP, unroll=4)
            def _(g):
                base = g * _L
                iv = idxb[pl.ds(cbase + base, _L)]
                obuf[sl, pl.ds(base, _L)] = plsc.load_gather(tslice, [iv])

            pltpu.async_copy(obuf.at[sl],
                             out_hbm.at[_N_NUM + f, d, pl.ds(cbase, _BC)], sem_o)
        pltpu.make_async_copy(obuf.at[0], out_hbm.at[0, 0, pl.ds(0, _BC)],
                              sem_o0).wait()
        pltpu.make_async_copy(obuf.at[1], out_hbm.at[0, 0, pl.ds(0, _BC)],
                              sem_o1).wait()
        return f

    lax.fori_loop(0, _CAT_JOBS_PER_W, cat_iter, jnp.int32(-1))


@jax.jit
def _sc_tokenize(xnum_t, xcat_t, wflat, bflat, tab_t):
    mesh = plsc.VectorSubcoreMesh(core_axis_name="c", subcore_axis_name="s")
    k = pl.kernel(
        _sc_body,
        out_type=jax.ShapeDtypeStruct((_TOK, _D, _B), jnp.float32),
        mesh=mesh,
        compiler_params=pltpu.CompilerParams(needs_layout_passes=False),
        scratch_types=[
            pltpu.VMEM((_VOCAB,), jnp.float32),
            pltpu.VMEM((_B,), jnp.int32),
            pltpu.VMEM((2, _BC), jnp.float32),
            pltpu.VMEM((2, _NBC), jnp.float32),
            pltpu.VMEM((_NUM_JOBS,), jnp.float32),
            pltpu.VMEM((_NUM_JOBS,), jnp.float32),
            pltpu.SemaphoreType.DMA,
            pltpu.SemaphoreType.DMA,
            pltpu.SemaphoreType.DMA,
            pltpu.SemaphoreType.DMA,
            pltpu.SemaphoreType.DMA,
            pltpu.SemaphoreType.DMA,
        ],
    )
    return k(xnum_t, xcat_t, wflat, bflat, tab_t)


def kernel(x_num, x_cat, W_num, b_num, tables):
    xnum_t = x_num.T
    xcat_t = x_cat.T.astype(jnp.int32)
    tab_t = jnp.transpose(tables, (0, 2, 1))
    wflat = W_num.reshape(-1)
    bflat = b_num.reshape(-1)
    out_t = _sc_tokenize(xnum_t, xcat_t, wflat, bflat, tab_t)
    return jnp.transpose(out_t, (2, 0, 1))


# speedup vs baseline: 14.2236x; 1.1914x over previous
import jax
import jax.numpy as jnp
from jax import lax
from jax.experimental import pallas as pl
from jax.experimental.pallas import tpu as pltpu
from jax.experimental.pallas import tpu_sc as plsc

_B = 16384
_N_NUM = 13
_N_CAT = 26
_VOCAB = 100000
_D = 16
_L = 16
_TOK = _N_NUM + _N_CAT

_NC = 2
_NS = 16
_NW = _NC * _NS

_CAT_JOBS_PER_W = (_N_CAT * _D) // _NW
_NUM_JOBS = _N_NUM * _D

_BC = 4096
_NCH = _B // _BC
_GRP = _BC // _L
_NBC = 2048
_NNCH = _B // _NBC
_NGRP = _NBC // _L


def _sc_body(xnum_hbm, xcat_hbm, wflat_hbm, bflat_hbm, tab_hbm, out_hbm,
             tslice, idxb, obuf, nbuf, wv, bv,
             sem_t, sem_i, sem_o0, sem_o1, sem_n0, sem_n1):
    wid = lax.axis_index("s") * _NC + lax.axis_index("c")

    pltpu.sync_copy(wflat_hbm, wv)
    pltpu.sync_copy(bflat_hbm, bv)

    def num_chunk(c):
        xsl = obuf.at[0, pl.ds(0, _NBC)]
        pltpu.sync_copy(xnum_hbm.at[wid, pl.ds(c * _NBC, _NBC)], xsl)
        for dd in range(_D):
            sl = dd % 2
            sem_n = sem_n0 if sl == 0 else sem_n1
            if dd >= 2:
                pltpu.make_async_copy(nbuf.at[sl], out_hbm.at[0, 0, pl.ds(0, _NBC)],
                                      sem_n).wait()
            sel = lax.broadcast_in_dim(wid * _D + dd, (_L,), ())
            ws = plsc.load_gather(wv, [sel])
            bs = plsc.load_gather(bv, [sel])

            @plsc.parallel_loop(0, _NGRP, unroll=4)
            def _(g):
                base = g * _L
                nbuf[sl, pl.ds(base, _L)] = obuf[0, pl.ds(base, _L)] * ws + bs

            pltpu.async_copy(nbuf.at[sl], out_hbm.at[wid, dd, pl.ds(c * _NBC, _NBC)],
                             sem_n)
        pltpu.make_async_copy(nbuf.at[0], out_hbm.at[0, 0, pl.ds(0, _NBC)],
                              sem_n0).wait()
        pltpu.make_async_copy(nbuf.at[1], out_hbm.at[0, 0, pl.ds(0, _NBC)],
                              sem_n1).wait()

    def cat_iter(i, prev_f):
        j = _CAT_JOBS_PER_W * wid + i
        f = lax.div(j, _D)
        d = lax.rem(j, _D)

        pltpu.async_copy(tab_hbm.at[f, d, :], tslice, sem_t)
        new_f = f != prev_f

        @pl.when(new_f)
        def _():
            pltpu.async_copy(xcat_hbm.at[f, :], idxb, sem_i)

        @pl.when(jnp.logical_and(wid < _N_NUM, i < _NNCH))
        def _():
            num_chunk(i)

        @pl.when(new_f)
        def _():
            pltpu.make_async_copy(xcat_hbm.at[f, :], idxb, sem_i).wait()

        pltpu.make_async_copy(tab_hbm.at[f, d, :], tslice, sem_t).wait()

        for c in range(_NCH):
            sl = c % 2
            sem_o = sem_o0 if sl == 0 else sem_o1
            if c >= 2:
                pltpu.make_async_copy(obuf.at[sl], out_hbm.at[0, 0, pl.ds(0, _BC)],
                                      sem_o).wait()
            cbase = c * _BC

            @plsc.parallel_loop(0, _GRP, unroll=4)
            def _(g):
                base = g * _L
                iv = idxb[pl.ds(cbase + base, _L)]
                obuf[sl, pl.ds(base, _L)] = plsc.load_gather(tslice, [iv])

            pltpu.async_copy(obuf.at[sl],
                             out_hbm.at[_N_NUM + f, d, pl.ds(cbase, _BC)], sem_o)
        pltpu.make_async_copy(obuf.at[0], out_hbm.at[0, 0, pl.ds(0, _BC)],
                              sem_o0).wait()
        pltpu.make_async_copy(obuf.at[1], out_hbm.at[0, 0, pl.ds(0, _BC)],
                              sem_o1).wait()
        return f

    lax.fori_loop(0, _CAT_JOBS_PER_W, cat_iter, jnp.int32(-1))


@jax.jit
def _sc_tokenize(xnum_t, xcat_t, wflat, bflat, tab_t):
    mesh = plsc.VectorSubcoreMesh(core_axis_name="c", subcore_axis_name="s")
    k = pl.kernel(
        _sc_body,
        out_type=jax.ShapeDtypeStruct((_TOK, _D, _B), jnp.float32),
        mesh=mesh,
        compiler_params=pltpu.CompilerParams(needs_layout_passes=False),
        scratch_types=[
            pltpu.VMEM((_VOCAB,), jnp.float32),
            pltpu.VMEM((_B,), jnp.int32),
            pltpu.VMEM((2, _BC), jnp.float32),
            pltpu.VMEM((2, _NBC), jnp.float32),
            pltpu.VMEM((_NUM_JOBS,), jnp.float32),
            pltpu.VMEM((_NUM_JOBS,), jnp.float32),
            pltpu.SemaphoreType.DMA,
            pltpu.SemaphoreType.DMA,
            pltpu.SemaphoreType.DMA,
            pltpu.SemaphoreType.DMA,
            pltpu.SemaphoreType.DMA,
            pltpu.SemaphoreType.DMA,
        ],
    )
    return k(xnum_t, xcat_t, wflat, bflat, tab_t)


def kernel(x_num, x_cat, W_num, b_num, tables):
    xnum_t = x_num.T
    xcat_t = x_cat.T.astype(jnp.int32)
    tab_t = jnp.transpose(tables, (0, 2, 1))
    wflat = W_num.reshape(-1)
    bflat = b_num.reshape(-1)
    out_t = _sc_tokenize(xnum_t, xcat_t, wflat, bflat, tab_t)
    return jnp.transpose(out_t, (2, 0, 1))
